# baseline (speedup 1.0000x reference)
import functools

import jax
import jax.numpy as jnp
from jax import lax
from jax.experimental import pallas as pl
from jax.experimental.pallas import tpu as pltpu
from jax.experimental.pallas import tpu_sc as plsc

N = 10000
E = 320000
D = 128
R = 16

NP = 10240
NW = 32
B = 128
CH = 16
TOTBLK = 2560
NB0 = 160
NB1 = 0
EPAD = TOTBLK * B
KT = 160768
KSLICE = KT // 16
SENT = N * R
ROWS_PER_TILE = NP // 16


def _zero_vmem_2d(buf, nrows):
    def body(i, _):
        for q in range(D // 16):
            buf[i, pl.ds(q * 16, 16)] = jnp.zeros((16,), jnp.float32)
        return 0
    lax.fori_loop(0, nrows, body, 0)


def _count_body(k2_hbm, out_hbm, kv, zb, ones, cbuf, cnt_sh, sem):
    c = lax.axis_index("c")
    s = lax.axis_index("s")
    w = s * 2 + c
    def zb_body(i, _):
        zb[pl.ds(i * 16, 16)] = jnp.zeros((16,), jnp.float32)
        return 0
    lax.fori_loop(0, 128, zb_body, 0)
    def ones_body(i, _):
        ones[pl.ds(i * 16, 16)] = jnp.ones((16,), jnp.float32)
        return 0
    lax.fori_loop(0, 8, ones_body, 0)
    base = pl.multiple_of(s * KSLICE, 8)
    for q in range(4):
        pltpu.sync_copy(zb, cnt_sh.at[pl.ds(base + q * 2048, 2048)])
    pltpu.sync_copy(zb.at[pl.ds(0, 1856)], cnt_sh.at[pl.ds(base + 8192, 1856)])
    plsc.subcore_barrier()
    def chunk(cc, _):
        cb = pl.multiple_of(w * (TOTBLK // NW) + cc * CH, 8)
        pltpu.sync_copy(k2_hbm.at[pl.ds(cb, CH)], kv)

        def blk(j, _):
            pltpu.sync_copy(ones, cnt_sh.at[kv.at[j]], add=True)
            return 0
        lax.fori_loop(0, CH, blk, 0)
        return 0
    lax.fori_loop(0, TOTBLK // NW // CH, chunk, 0)
    plsc.subcore_barrier()
    obase = pl.multiple_of(c * KT + base, 8)
    pltpu.sync_copy(cnt_sh.at[pl.ds(base, KSLICE)], cbuf)
    pltpu.sync_copy(cbuf, out_hbm.at[pl.ds(obase, KSLICE)])


_count_kernel = functools.partial(
    pl.kernel,
    out_type=jax.ShapeDtypeStruct((2 * KT,), jnp.float32),
    mesh=plsc.VectorSubcoreMesh(core_axis_name="c", subcore_axis_name="s",
                                num_cores=2, num_subcores=16),
    compiler_params=pltpu.CompilerParams(needs_layout_passes=False),
    scratch_types=[
        pltpu.VMEM((CH, B), jnp.int32),
        pltpu.VMEM((2048,), jnp.float32),
        pltpu.VMEM((B,), jnp.float32),
        pltpu.VMEM((KSLICE,), jnp.float32),
        pltpu.VMEM_SHARED((KT,), jnp.float32),
        pltpu.SemaphoreType.DMA,
    ],
)(_count_body)


def _inv_body(cnt_ref, out_ref):
    c = cnt_ref[0] + cnt_ref[1]
    row = lax.broadcasted_iota(jnp.int32, (KT // 128, 128), 0)
    col = lax.broadcasted_iota(jnp.int32, (KT // 128, 128), 1)
    valid = (row * 128 + col) < SENT
    out_ref[...] = jnp.where(valid, 1.0 / jnp.maximum(c, 1.0), 0.0)


def _inv_counts(cnt):
    out = pl.pallas_call(
        _inv_body,
        out_shape=jax.ShapeDtypeStruct((KT // 128, 128), jnp.float32),
    )(cnt.reshape(2, KT // 128, 128))
    return out.reshape(KT)


BN = 512


def _mm_body(x_ref, w_ref, b_ref, out_ref):
    r = pl.program_id(0)
    acc = jnp.dot(x_ref[...], w_ref[0], preferred_element_type=jnp.float32)
    acc = acc + jnp.where(r == R, 1.0, 0.0) * b_ref[0]
    out_ref[0] = acc


def _transform(xp, Ws, b):
    return pl.pallas_call(
        _mm_body,
        grid=(R + 1,),
        in_specs=[
            pl.BlockSpec((NP, D), lambda r: (0, 0)),
            pl.BlockSpec((1, D, D), lambda r: (r, 0, 0)),
            pl.BlockSpec((1, D), lambda r: (0, 0)),
        ],
        out_specs=pl.BlockSpec((1, NP, D), lambda r: (r, 0, 0)),
        out_shape=jax.ShapeDtypeStruct((R + 1, NP, D), jnp.float32),
    )(xp, Ws, b.reshape(1, D))


def _scale_block(rows_p, sv_p):
    def edge(b, _):
        sc = plsc.load_gather(sv_p, [jnp.full((16,), b, jnp.int32)])
        for q in range(D // 16):
            rows_p[b, pl.ds(q * 16, 16)] = rows_p[b, pl.ds(q * 16, 16)] * sc
        return 0
    lax.fori_loop(0, B, edge, 0)


def _agg_body(h_hbm, g2_hbm, k2_hbm, d2_hbm, inv_hbm, out_hbm,
              gq, kq, dq, rows, sv, zbuf,
              acc, sem_g0, sem_g1, sem_s0, sem_s1, sem_w0, sem_w1):
    c = lax.axis_index("c")
    s = lax.axis_index("s")
    sem_g = (sem_g0, sem_g1)
    sem_s = (sem_s0, sem_s1)
    sem_w = (sem_w0, sem_w1)
    _zero_vmem_2d(zbuf, 16)
    rbase = pl.multiple_of(s * ROWS_PER_TILE, 8)
    for q in range(ROWS_PER_TILE // 16):
        pltpu.sync_copy(zbuf, acc.at[pl.ds(rbase + q * 16, 16)])
    plsc.subcore_barrier()

    nchunks = jnp.where(c == 0, NB0 // CH, NB1 // CH)
    start = jnp.where(c == 0, s * NB0, 16 * NB0 + s * NB1)

    def chunk(cc, _):
        cb = pl.multiple_of(start + cc * CH, 8)
        pltpu.sync_copy(g2_hbm.at[pl.ds(cb, CH)], gq)
        pltpu.sync_copy(k2_hbm.at[pl.ds(cb, CH)], kq)
        pltpu.sync_copy(d2_hbm.at[pl.ds(cb, CH)], dq)

        g_descs = [None, None]
        s_descs = [None, None]
        w_descs = [None, None]
        g_descs[0] = pltpu.async_copy(h_hbm.at[gq.at[0]], rows.at[0], sem_g[0])
        s_descs[0] = pltpu.async_copy(inv_hbm.at[kq.at[0]], sv.at[0], sem_s[0])
        for j in range(CH):
            p = j % 2
            q = 1 - p
            g_descs[p].wait()
            s_descs[p].wait()
            if j + 1 < CH:
                if j >= 1:
                    w_descs[q].wait()
                    w_descs[q] = None
                g_descs[q] = pltpu.async_copy(
                    h_hbm.at[gq.at[j + 1]], rows.at[q], sem_g[q])
                s_descs[q] = pltpu.async_copy(
                    inv_hbm.at[kq.at[j + 1]], sv.at[q], sem_s[q])
            _scale_block(rows.at[p], sv.at[p])
            w_descs[p] = pltpu.async_copy(
                rows.at[p], acc.at[dq.at[j]], sem_w[p], add=True)
        for d in w_descs:
            if d is not None:
                d.wait()
        return 0
    lax.fori_loop(0, nchunks, chunk, 0)
    plsc.subcore_barrier()
    for q in range(ROWS_PER_TILE // B):
        pltpu.sync_copy(acc.at[pl.ds(rbase + q * B, B)],
                        out_hbm.at[c, pl.ds(rbase + q * B, B)])


_agg_kernel = functools.partial(
    pl.kernel,
    out_type=jax.ShapeDtypeStruct((2, NP, D), jnp.float32),
    mesh=plsc.VectorSubcoreMesh(core_axis_name="c", subcore_axis_name="s",
                                num_cores=2, num_subcores=16),
    compiler_params=pltpu.CompilerParams(needs_layout_passes=False),
    scratch_types=[
        pltpu.VMEM((CH, B), jnp.int32),
        pltpu.VMEM((CH, B), jnp.int32),
        pltpu.VMEM((CH, B), jnp.int32),
        pltpu.VMEM((2, B, D), jnp.float32),
        pltpu.VMEM((2, B), jnp.float32),
        pltpu.VMEM((16, D), jnp.float32),
        pltpu.VMEM_SHARED((NP, D), jnp.float32),
        pltpu.SemaphoreType.DMA,
        pltpu.SemaphoreType.DMA,
        pltpu.SemaphoreType.DMA,
        pltpu.SemaphoreType.DMA,
        pltpu.SemaphoreType.DMA,
        pltpu.SemaphoreType.DMA,
    ],
)(_agg_body)


def _comb_body(p_ref, r_ref, o_ref):
    o_ref[...] = jnp.maximum(p_ref[0] + p_ref[1] + r_ref[...], 0.0)


def _combine(parts, root):
    return pl.pallas_call(
        _comb_body,
        grid=(NP // BN,),
        in_specs=[
            pl.BlockSpec((2, BN, D), lambda j: (0, j, 0)),
            pl.BlockSpec((BN, D), lambda j: (j, 0)),
        ],
        out_specs=pl.BlockSpec((BN, D), lambda j: (j, 0)),
        out_shape=jax.ShapeDtypeStruct((NP, D), jnp.float32),
    )(parts, root)


def _layer(xp, Ws, b, g3, k3, d3, inv):
    h = _transform(xp, Ws, b)
    h_flat = h.reshape((R + 1) * NP, D)
    parts = _agg_kernel(h_flat, g3, k3, d3, inv)
    return _combine(parts, h[R])


def kernel(x, edge_index, edge_type, W1, Wroot1, b1, W2, Wroot2, b2):
    src = edge_index[0].astype(jnp.int32)
    dst = edge_index[1].astype(jnp.int32)
    et = edge_type.astype(jnp.int32)

    g = et * NP + src
    k = dst * R + et
    pad = EPAD - E
    g3 = jnp.pad(g, (0, pad)).reshape(TOTBLK, B)
    k3 = jnp.pad(k, (0, pad), constant_values=SENT).reshape(TOTBLK, B)
    d3 = jnp.pad(dst, (0, pad)).reshape(TOTBLK, B)

    xp = jnp.pad(x, ((0, NP - N), (0, 0)))
    W1s = jnp.concatenate([W1, Wroot1[None]], axis=0)
    W2s = jnp.concatenate([W2, Wroot2[None]], axis=0)

    cnt = _count_kernel(k3).reshape(2, KT)
    inv = _inv_counts(cnt)

    h1 = _layer(xp, W1s, b1, g3, k3, d3, inv)
    h2 = _layer(h1, W2s, b2, g3, k3, d3, inv)
    return h2[:N]


# speedup vs baseline: 1.3426x; 1.3426x over previous
import functools

import jax
import jax.numpy as jnp
from jax import lax
from jax.experimental import pallas as pl
from jax.experimental.pallas import tpu as pltpu
from jax.experimental.pallas import tpu_sc as plsc

N = 10000
E = 320000
D = 128
R = 16

NP = 10240
NW = 32
B = 128
CH = 16
TOTBLK = 2560
NB0 = 96
NB1 = 64
EPAD = TOTBLK * B
KT = 160768
KSLICE = KT // 16
SENT = N * R
ROWS_PER_TILE = NP // 16


def _zero_vmem_2d(buf, nrows):
    def body(i, _):
        for q in range(D // 16):
            buf[i, pl.ds(q * 16, 16)] = jnp.zeros((16,), jnp.float32)
        return 0
    lax.fori_loop(0, nrows, body, 0)


def _count_body(k2_hbm, out_hbm, kv, zb, ones, cbuf, cnt_sh, sem):
    c = lax.axis_index("c")
    s = lax.axis_index("s")
    w = s * 2 + c
    def zb_body(i, _):
        zb[pl.ds(i * 16, 16)] = jnp.zeros((16,), jnp.float32)
        return 0
    lax.fori_loop(0, 128, zb_body, 0)
    def ones_body(i, _):
        ones[pl.ds(i * 16, 16)] = jnp.ones((16,), jnp.float32)
        return 0
    lax.fori_loop(0, 8, ones_body, 0)
    base = pl.multiple_of(s * KSLICE, 8)
    for q in range(4):
        pltpu.sync_copy(zb, cnt_sh.at[pl.ds(base + q * 2048, 2048)])
    pltpu.sync_copy(zb.at[pl.ds(0, 1856)], cnt_sh.at[pl.ds(base + 8192, 1856)])
    plsc.subcore_barrier()
    def chunk(cc, _):
        cb = pl.multiple_of(w * (TOTBLK // NW) + cc * CH, 8)
        pltpu.sync_copy(k2_hbm.at[pl.ds(cb, CH)], kv)

        def blk(j, _):
            pltpu.sync_copy(ones, cnt_sh.at[kv.at[j]], add=True)
            return 0
        lax.fori_loop(0, CH, blk, 0)
        return 0
    lax.fori_loop(0, TOTBLK // NW // CH, chunk, 0)
    plsc.subcore_barrier()
    obase = pl.multiple_of(c * KT + base, 8)
    pltpu.sync_copy(cnt_sh.at[pl.ds(base, KSLICE)], cbuf)
    pltpu.sync_copy(cbuf, out_hbm.at[pl.ds(obase, KSLICE)])


_count_kernel = functools.partial(
    pl.kernel,
    out_type=jax.ShapeDtypeStruct((2 * KT,), jnp.float32),
    mesh=plsc.VectorSubcoreMesh(core_axis_name="c", subcore_axis_name="s",
                                num_cores=2, num_subcores=16),
    compiler_params=pltpu.CompilerParams(needs_layout_passes=False),
    scratch_types=[
        pltpu.VMEM((CH, B), jnp.int32),
        pltpu.VMEM((2048,), jnp.float32),
        pltpu.VMEM((B,), jnp.float32),
        pltpu.VMEM((KSLICE,), jnp.float32),
        pltpu.VMEM_SHARED((KT,), jnp.float32),
        pltpu.SemaphoreType.DMA,
    ],
)(_count_body)


def _inv_body(cnt_ref, out_ref):
    c = cnt_ref[0] + cnt_ref[1]
    row = lax.broadcasted_iota(jnp.int32, (KT // 128, 128), 0)
    col = lax.broadcasted_iota(jnp.int32, (KT // 128, 128), 1)
    valid = (row * 128 + col) < SENT
    out_ref[...] = jnp.where(valid, 1.0 / jnp.maximum(c, 1.0), 0.0)


def _inv_counts(cnt):
    out = pl.pallas_call(
        _inv_body,
        out_shape=jax.ShapeDtypeStruct((KT // 128, 128), jnp.float32),
    )(cnt.reshape(2, KT // 128, 128))
    return out.reshape(KT)


BN = 512


def _mm_body(x_ref, w_ref, b_ref, out_ref):
    r = pl.program_id(0)
    acc = jnp.dot(x_ref[...], w_ref[0], preferred_element_type=jnp.float32)
    acc = acc + jnp.where(r == R, 1.0, 0.0) * b_ref[0]
    out_ref[0] = acc


def _transform(xp, Ws, b):
    return pl.pallas_call(
        _mm_body,
        grid=(R + 1,),
        in_specs=[
            pl.BlockSpec((NP, D), lambda r: (0, 0)),
            pl.BlockSpec((1, D, D), lambda r: (r, 0, 0)),
            pl.BlockSpec((1, D), lambda r: (0, 0)),
        ],
        out_specs=pl.BlockSpec((1, NP, D), lambda r: (r, 0, 0)),
        out_shape=jax.ShapeDtypeStruct((R + 1, NP, D), jnp.float32),
    )(xp, Ws, b.reshape(1, D))


def _scale_block(rows_p, sv_p):
    def edge(b, _):
        sc = plsc.load_gather(sv_p, [jnp.full((16,), b, jnp.int32)])
        for q in range(D // 16):
            rows_p[b, pl.ds(q * 16, 16)] = rows_p[b, pl.ds(q * 16, 16)] * sc
        return 0
    lax.fori_loop(0, B, edge, 0)


def _agg_body(h_hbm, g2_hbm, k2_hbm, d2_hbm, inv_hbm, out_hbm,
              gq, kq, dq, rows, sv, zbuf,
              acc, sem_g0, sem_g1, sem_s0, sem_s1, sem_w0, sem_w1):
    c = lax.axis_index("c")
    s = lax.axis_index("s")
    sem_g = (sem_g0, sem_g1)
    sem_s = (sem_s0, sem_s1)
    sem_w = (sem_w0, sem_w1)
    _zero_vmem_2d(zbuf, 16)
    rbase = pl.multiple_of(s * ROWS_PER_TILE, 8)
    for q in range(ROWS_PER_TILE // 16):
        pltpu.sync_copy(zbuf, acc.at[pl.ds(rbase + q * 16, 16)])
    plsc.subcore_barrier()

    nchunks = jnp.where(c == 0, NB0 // CH, NB1 // CH)
    start = jnp.where(c == 0, s * NB0, 16 * NB0 + s * NB1)

    def chunk(cc, _):
        cb = pl.multiple_of(start + cc * CH, 8)
        pltpu.sync_copy(g2_hbm.at[pl.ds(cb, CH)], gq)
        pltpu.sync_copy(k2_hbm.at[pl.ds(cb, CH)], kq)
        pltpu.sync_copy(d2_hbm.at[pl.ds(cb, CH)], dq)

        g_descs = [None, None]
        s_descs = [None, None]
        w_descs = [None, None]
        g_descs[0] = pltpu.async_copy(h_hbm.at[gq.at[0]], rows.at[0], sem_g[0])
        s_descs[0] = pltpu.async_copy(inv_hbm.at[kq.at[0]], sv.at[0], sem_s[0])
        for j in range(CH):
            p = j % 2
            q = 1 - p
            g_descs[p].wait()
            s_descs[p].wait()
            if j + 1 < CH:
                if j >= 1:
                    w_descs[q].wait()
                    w_descs[q] = None
                g_descs[q] = pltpu.async_copy(
                    h_hbm.at[gq.at[j + 1]], rows.at[q], sem_g[q])
                s_descs[q] = pltpu.async_copy(
                    inv_hbm.at[kq.at[j + 1]], sv.at[q], sem_s[q])
            _scale_block(rows.at[p], sv.at[p])
            w_descs[p] = pltpu.async_copy(
                rows.at[p], acc.at[dq.at[j]], sem_w[p], add=True)
        for d in w_descs:
            if d is not None:
                d.wait()
        return 0
    lax.fori_loop(0, nchunks, chunk, 0)
    plsc.subcore_barrier()
    for q in range(ROWS_PER_TILE // B):
        pltpu.sync_copy(acc.at[pl.ds(rbase + q * B, B)],
                        out_hbm.at[c, pl.ds(rbase + q * B, B)])


_agg_kernel = functools.partial(
    pl.kernel,
    out_type=jax.ShapeDtypeStruct((2, NP, D), jnp.float32),
    mesh=plsc.VectorSubcoreMesh(core_axis_name="c", subcore_axis_name="s",
                                num_cores=2, num_subcores=16),
    compiler_params=pltpu.CompilerParams(needs_layout_passes=False),
    scratch_types=[
        pltpu.VMEM((CH, B), jnp.int32),
        pltpu.VMEM((CH, B), jnp.int32),
        pltpu.VMEM((CH, B), jnp.int32),
        pltpu.VMEM((2, B, D), jnp.float32),
        pltpu.VMEM((2, B), jnp.float32),
        pltpu.VMEM((16, D), jnp.float32),
        pltpu.VMEM_SHARED((NP, D), jnp.float32),
        pltpu.SemaphoreType.DMA,
        pltpu.SemaphoreType.DMA,
        pltpu.SemaphoreType.DMA,
        pltpu.SemaphoreType.DMA,
        pltpu.SemaphoreType.DMA,
        pltpu.SemaphoreType.DMA,
    ],
)(_agg_body)


def _comb_body(p_ref, r_ref, o_ref):
    o_ref[...] = jnp.maximum(p_ref[0] + p_ref[1] + r_ref[...], 0.0)


def _combine(parts, root):
    return pl.pallas_call(
        _comb_body,
        grid=(NP // BN,),
        in_specs=[
            pl.BlockSpec((2, BN, D), lambda j: (0, j, 0)),
            pl.BlockSpec((BN, D), lambda j: (j, 0)),
        ],
        out_specs=pl.BlockSpec((BN, D), lambda j: (j, 0)),
        out_shape=jax.ShapeDtypeStruct((NP, D), jnp.float32),
    )(parts, root)


def _layer(xp, Ws, b, g3, k3, d3, inv):
    h = _transform(xp, Ws, b)
    h_flat = h.reshape((R + 1) * NP, D)
    parts = _agg_kernel(h_flat, g3, k3, d3, inv)
    return _combine(parts, h[R])


def kernel(x, edge_index, edge_type, W1, Wroot1, b1, W2, Wroot2, b2):
    src = edge_index[0].astype(jnp.int32)
    dst = edge_index[1].astype(jnp.int32)
    et = edge_type.astype(jnp.int32)

    g = et * NP + src
    k = dst * R + et
    pad = EPAD - E
    g3 = jnp.pad(g, (0, pad)).reshape(TOTBLK, B)
    k3 = jnp.pad(k, (0, pad), constant_values=SENT).reshape(TOTBLK, B)
    d3 = jnp.pad(dst, (0, pad)).reshape(TOTBLK, B)

    xp = jnp.pad(x, ((0, NP - N), (0, 0)))
    W1s = jnp.concatenate([W1, Wroot1[None]], axis=0)
    W2s = jnp.concatenate([W2, Wroot2[None]], axis=0)

    cnt = _count_kernel(k3).reshape(2, KT)
    inv = _inv_counts(cnt)

    h1 = _layer(xp, W1s, b1, g3, k3, d3, inv)
    h2 = _layer(h1, W2s, b2, g3, k3, d3, inv)
    return h2[:N]


# speedup vs baseline: 1.4535x; 1.0826x over previous
import functools

import jax
import jax.numpy as jnp
from jax import lax
from jax.experimental import pallas as pl
from jax.experimental.pallas import tpu as pltpu
from jax.experimental.pallas import tpu_sc as plsc

N = 10000
E = 320000
D = 128
R = 16

NP = 10240
NW = 32
B = 128
CH = 16
TOTBLK = 2560
NB0 = 128
NB1 = 32
EPAD = TOTBLK * B
KT = 160768
KSLICE = KT // 16
SENT = N * R
ROWS_PER_TILE = NP // 16


def _zero_vmem_2d(buf, nrows):
    def body(i, _):
        for q in range(D // 16):
            buf[i, pl.ds(q * 16, 16)] = jnp.zeros((16,), jnp.float32)
        return 0
    lax.fori_loop(0, nrows, body, 0)


def _count_body(k2_hbm, out_hbm, kv, zb, ones, cbuf, cnt_sh, sem):
    c = lax.axis_index("c")
    s = lax.axis_index("s")
    w = s * 2 + c
    def zb_body(i, _):
        zb[pl.ds(i * 16, 16)] = jnp.zeros((16,), jnp.float32)
        return 0
    lax.fori_loop(0, 128, zb_body, 0)
    def ones_body(i, _):
        ones[pl.ds(i * 16, 16)] = jnp.ones((16,), jnp.float32)
        return 0
    lax.fori_loop(0, 8, ones_body, 0)
    base = pl.multiple_of(s * KSLICE, 8)
    for q in range(4):
        pltpu.sync_copy(zb, cnt_sh.at[pl.ds(base + q * 2048, 2048)])
    pltpu.sync_copy(zb.at[pl.ds(0, 1856)], cnt_sh.at[pl.ds(base + 8192, 1856)])
    plsc.subcore_barrier()
    def chunk(cc, _):
        cb = pl.multiple_of(w * (TOTBLK // NW) + cc * CH, 8)
        pltpu.sync_copy(k2_hbm.at[pl.ds(cb, CH)], kv)

        def blk(j, _):
            pltpu.sync_copy(ones, cnt_sh.at[kv.at[j]], add=True)
            return 0
        lax.fori_loop(0, CH, blk, 0)
        return 0
    lax.fori_loop(0, TOTBLK // NW // CH, chunk, 0)
    plsc.subcore_barrier()
    obase = pl.multiple_of(c * KT + base, 8)
    pltpu.sync_copy(cnt_sh.at[pl.ds(base, KSLICE)], cbuf)
    pltpu.sync_copy(cbuf, out_hbm.at[pl.ds(obase, KSLICE)])


_count_kernel = functools.partial(
    pl.kernel,
    out_type=jax.ShapeDtypeStruct((2 * KT,), jnp.float32),
    mesh=plsc.VectorSubcoreMesh(core_axis_name="c", subcore_axis_name="s",
                                num_cores=2, num_subcores=16),
    compiler_params=pltpu.CompilerParams(needs_layout_passes=False),
    scratch_types=[
        pltpu.VMEM((CH, B), jnp.int32),
        pltpu.VMEM((2048,), jnp.float32),
        pltpu.VMEM((B,), jnp.float32),
        pltpu.VMEM((KSLICE,), jnp.float32),
        pltpu.VMEM_SHARED((KT,), jnp.float32),
        pltpu.SemaphoreType.DMA,
    ],
)(_count_body)


def _inv_body(cnt_ref, out_ref):
    c = cnt_ref[0] + cnt_ref[1]
    row = lax.broadcasted_iota(jnp.int32, (KT // 128, 128), 0)
    col = lax.broadcasted_iota(jnp.int32, (KT // 128, 128), 1)
    valid = (row * 128 + col) < SENT
    out_ref[...] = jnp.where(valid, 1.0 / jnp.maximum(c, 1.0), 0.0)


def _inv_counts(cnt):
    out = pl.pallas_call(
        _inv_body,
        out_shape=jax.ShapeDtypeStruct((KT // 128, 128), jnp.float32),
    )(cnt.reshape(2, KT // 128, 128))
    return out.reshape(KT)


BN = 512


def _mm_body(x_ref, w_ref, b_ref, out_ref):
    r = pl.program_id(0)
    acc = jnp.dot(x_ref[...], w_ref[0], preferred_element_type=jnp.float32)
    acc = acc + jnp.where(r == R, 1.0, 0.0) * b_ref[0]
    out_ref[0] = acc


def _transform(xp, Ws, b):
    return pl.pallas_call(
        _mm_body,
        grid=(R + 1,),
        in_specs=[
            pl.BlockSpec((NP, D), lambda r: (0, 0)),
            pl.BlockSpec((1, D, D), lambda r: (r, 0, 0)),
            pl.BlockSpec((1, D), lambda r: (0, 0)),
        ],
        out_specs=pl.BlockSpec((1, NP, D), lambda r: (r, 0, 0)),
        out_shape=jax.ShapeDtypeStruct((R + 1, NP, D), jnp.float32),
    )(xp, Ws, b.reshape(1, D))


def _scale_block(rows_p, sv_p):
    def edge(b, _):
        sc = plsc.load_gather(sv_p, [jnp.full((16,), b, jnp.int32)])
        for q in range(D // 16):
            rows_p[b, pl.ds(q * 16, 16)] = rows_p[b, pl.ds(q * 16, 16)] * sc
        return 0
    lax.fori_loop(0, B, edge, 0)


def _agg_body(h_hbm, g2_hbm, k2_hbm, d2_hbm, inv_hbm, out_hbm,
              gq, kq, dq, rows, sv, zbuf,
              acc, sem_g0, sem_g1, sem_s0, sem_s1, sem_w0, sem_w1):
    c = lax.axis_index("c")
    s = lax.axis_index("s")
    sem_g = (sem_g0, sem_g1)
    sem_s = (sem_s0, sem_s1)
    sem_w = (sem_w0, sem_w1)
    _zero_vmem_2d(zbuf, 16)
    rbase = pl.multiple_of(s * ROWS_PER_TILE, 8)
    for q in range(ROWS_PER_TILE // 16):
        pltpu.sync_copy(zbuf, acc.at[pl.ds(rbase + q * 16, 16)])
    plsc.subcore_barrier()

    nchunks = jnp.where(c == 0, NB0 // CH, NB1 // CH)
    start = jnp.where(c == 0, s * NB0, 16 * NB0 + s * NB1)

    def chunk(cc, _):
        cb = pl.multiple_of(start + cc * CH, 8)
        pltpu.sync_copy(g2_hbm.at[pl.ds(cb, CH)], gq)
        pltpu.sync_copy(k2_hbm.at[pl.ds(cb, CH)], kq)
        pltpu.sync_copy(d2_hbm.at[pl.ds(cb, CH)], dq)

        g_descs = [None, None]
        s_descs = [None, None]
        w_descs = [None, None]
        g_descs[0] = pltpu.async_copy(h_hbm.at[gq.at[0]], rows.at[0], sem_g[0])
        s_descs[0] = pltpu.async_copy(inv_hbm.at[kq.at[0]], sv.at[0], sem_s[0])
        for j in range(CH):
            p = j % 2
            q = 1 - p
            g_descs[p].wait()
            s_descs[p].wait()
            if j + 1 < CH:
                if j >= 1:
                    w_descs[q].wait()
                    w_descs[q] = None
                g_descs[q] = pltpu.async_copy(
                    h_hbm.at[gq.at[j + 1]], rows.at[q], sem_g[q])
                s_descs[q] = pltpu.async_copy(
                    inv_hbm.at[kq.at[j + 1]], sv.at[q], sem_s[q])
            _scale_block(rows.at[p], sv.at[p])
            w_descs[p] = pltpu.async_copy(
                rows.at[p], acc.at[dq.at[j]], sem_w[p], add=True)
        for d in w_descs:
            if d is not None:
                d.wait()
        return 0
    lax.fori_loop(0, nchunks, chunk, 0)
    plsc.subcore_barrier()
    for q in range(ROWS_PER_TILE // B):
        pltpu.sync_copy(acc.at[pl.ds(rbase + q * B, B)],
                        out_hbm.at[c, pl.ds(rbase + q * B, B)])


_agg_kernel = functools.partial(
    pl.kernel,
    out_type=jax.ShapeDtypeStruct((2, NP, D), jnp.float32),
    mesh=plsc.VectorSubcoreMesh(core_axis_name="c", subcore_axis_name="s",
                                num_cores=2, num_subcores=16),
    compiler_params=pltpu.CompilerParams(needs_layout_passes=False),
    scratch_types=[
        pltpu.VMEM((CH, B), jnp.int32),
        pltpu.VMEM((CH, B), jnp.int32),
        pltpu.VMEM((CH, B), jnp.int32),
        pltpu.VMEM((2, B, D), jnp.float32),
        pltpu.VMEM((2, B), jnp.float32),
        pltpu.VMEM((16, D), jnp.float32),
        pltpu.VMEM_SHARED((NP, D), jnp.float32),
        pltpu.SemaphoreType.DMA,
        pltpu.SemaphoreType.DMA,
        pltpu.SemaphoreType.DMA,
        pltpu.SemaphoreType.DMA,
        pltpu.SemaphoreType.DMA,
        pltpu.SemaphoreType.DMA,
    ],
)(_agg_body)


def _comb_body(p_ref, r_ref, o_ref):
    o_ref[...] = jnp.maximum(p_ref[0] + p_ref[1] + r_ref[...], 0.0)


def _combine(parts, root):
    return pl.pallas_call(
        _comb_body,
        grid=(NP // BN,),
        in_specs=[
            pl.BlockSpec((2, BN, D), lambda j: (0, j, 0)),
            pl.BlockSpec((BN, D), lambda j: (j, 0)),
        ],
        out_specs=pl.BlockSpec((BN, D), lambda j: (j, 0)),
        out_shape=jax.ShapeDtypeStruct((NP, D), jnp.float32),
    )(parts, root)


def _layer(xp, Ws, b, g3, k3, d3, inv):
    h = _transform(xp, Ws, b)
    h_flat = h.reshape((R + 1) * NP, D)
    parts = _agg_kernel(h_flat, g3, k3, d3, inv)
    return _combine(parts, h[R])


def kernel(x, edge_index, edge_type, W1, Wroot1, b1, W2, Wroot2, b2):
    src = edge_index[0].astype(jnp.int32)
    dst = edge_index[1].astype(jnp.int32)
    et = edge_type.astype(jnp.int32)

    g = et * NP + src
    k = dst * R + et
    pad = EPAD - E
    g3 = jnp.pad(g, (0, pad)).reshape(TOTBLK, B)
    k3 = jnp.pad(k, (0, pad), constant_values=SENT).reshape(TOTBLK, B)
    d3 = jnp.pad(dst, (0, pad)).reshape(TOTBLK, B)

    xp = jnp.pad(x, ((0, NP - N), (0, 0)))
    W1s = jnp.concatenate([W1, Wroot1[None]], axis=0)
    W2s = jnp.concatenate([W2, Wroot2[None]], axis=0)

    cnt = _count_kernel(k3).reshape(2, KT)
    inv = _inv_counts(cnt)

    h1 = _layer(xp, W1s, b1, g3, k3, d3, inv)
    h2 = _layer(h1, W2s, b2, g3, k3, d3, inv)
    return h2[:N]


# speedup vs baseline: 1.6430x; 1.1304x over previous
import functools

import jax
import jax.numpy as jnp
from jax import lax
from jax.experimental import pallas as pl
from jax.experimental.pallas import tpu as pltpu
from jax.experimental.pallas import tpu_sc as plsc

N = 10000
E = 320000
D = 128
R = 16

NP = 10240
NW = 32
B = 128
CH = 16
TOTBLK = 2560
NB0 = 144
NB1 = 16
EPAD = TOTBLK * B
KT = 160768
KSLICE = KT // 16
SENT = N * R
ROWS_PER_TILE = NP // 16


def _zero_vmem_2d(buf, nrows):
    def body(i, _):
        for q in range(D // 16):
            buf[i, pl.ds(q * 16, 16)] = jnp.zeros((16,), jnp.float32)
        return 0
    lax.fori_loop(0, nrows, body, 0)


def _count_body(k2_hbm, out_hbm, kv, zb, ones, cbuf, cnt_sh, sem):
    c = lax.axis_index("c")
    s = lax.axis_index("s")
    w = s * 2 + c
    def zb_body(i, _):
        zb[pl.ds(i * 16, 16)] = jnp.zeros((16,), jnp.float32)
        return 0
    lax.fori_loop(0, 128, zb_body, 0)
    def ones_body(i, _):
        ones[pl.ds(i * 16, 16)] = jnp.ones((16,), jnp.float32)
        return 0
    lax.fori_loop(0, 8, ones_body, 0)
    base = pl.multiple_of(s * KSLICE, 8)
    for q in range(4):
        pltpu.sync_copy(zb, cnt_sh.at[pl.ds(base + q * 2048, 2048)])
    pltpu.sync_copy(zb.at[pl.ds(0, 1856)], cnt_sh.at[pl.ds(base + 8192, 1856)])
    plsc.subcore_barrier()
    def chunk(cc, _):
        cb = pl.multiple_of(w * (TOTBLK // NW) + cc * CH, 8)
        pltpu.sync_copy(k2_hbm.at[pl.ds(cb, CH)], kv)

        def blk(j, _):
            pltpu.sync_copy(ones, cnt_sh.at[kv.at[j]], add=True)
            return 0
        lax.fori_loop(0, CH, blk, 0)
        return 0
    lax.fori_loop(0, TOTBLK // NW // CH, chunk, 0)
    plsc.subcore_barrier()
    obase = pl.multiple_of(c * KT + base, 8)
    pltpu.sync_copy(cnt_sh.at[pl.ds(base, KSLICE)], cbuf)
    pltpu.sync_copy(cbuf, out_hbm.at[pl.ds(obase, KSLICE)])


_count_kernel = functools.partial(
    pl.kernel,
    out_type=jax.ShapeDtypeStruct((2 * KT,), jnp.float32),
    mesh=plsc.VectorSubcoreMesh(core_axis_name="c", subcore_axis_name="s",
                                num_cores=2, num_subcores=16),
    compiler_params=pltpu.CompilerParams(needs_layout_passes=False),
    scratch_types=[
        pltpu.VMEM((CH, B), jnp.int32),
        pltpu.VMEM((2048,), jnp.float32),
        pltpu.VMEM((B,), jnp.float32),
        pltpu.VMEM((KSLICE,), jnp.float32),
        pltpu.VMEM_SHARED((KT,), jnp.float32),
        pltpu.SemaphoreType.DMA,
    ],
)(_count_body)


def _inv_body(cnt_ref, out_ref):
    c = cnt_ref[0] + cnt_ref[1]
    row = lax.broadcasted_iota(jnp.int32, (KT // 128, 128), 0)
    col = lax.broadcasted_iota(jnp.int32, (KT // 128, 128), 1)
    valid = (row * 128 + col) < SENT
    out_ref[...] = jnp.where(valid, 1.0 / jnp.maximum(c, 1.0), 0.0)


def _inv_counts(cnt):
    out = pl.pallas_call(
        _inv_body,
        out_shape=jax.ShapeDtypeStruct((KT // 128, 128), jnp.float32),
    )(cnt.reshape(2, KT // 128, 128))
    return out.reshape(KT)


BN = 512


def _mm_body(x_ref, w_ref, b_ref, out_ref):
    r = pl.program_id(0)
    acc = jnp.dot(x_ref[...], w_ref[0], preferred_element_type=jnp.float32)
    acc = acc + jnp.where(r == R, 1.0, 0.0) * b_ref[0]
    out_ref[0] = acc


def _transform(xp, Ws, b):
    return pl.pallas_call(
        _mm_body,
        grid=(R + 1,),
        in_specs=[
            pl.BlockSpec((NP, D), lambda r: (0, 0)),
            pl.BlockSpec((1, D, D), lambda r: (r, 0, 0)),
            pl.BlockSpec((1, D), lambda r: (0, 0)),
        ],
        out_specs=pl.BlockSpec((1, NP, D), lambda r: (r, 0, 0)),
        out_shape=jax.ShapeDtypeStruct((R + 1, NP, D), jnp.float32),
    )(xp, Ws, b.reshape(1, D))


def _scale_block(rows_p, sv_p):
    def edge(b, _):
        sc = plsc.load_gather(sv_p, [jnp.full((16,), b, jnp.int32)])
        for q in range(D // 16):
            rows_p[b, pl.ds(q * 16, 16)] = rows_p[b, pl.ds(q * 16, 16)] * sc
        return 0
    lax.fori_loop(0, B, edge, 0)


def _agg_body(h_hbm, g2_hbm, k2_hbm, d2_hbm, inv_hbm, out_hbm,
              gq, kq, dq, rows, sv, zbuf,
              acc, sem_g0, sem_g1, sem_s0, sem_s1, sem_w0, sem_w1):
    c = lax.axis_index("c")
    s = lax.axis_index("s")
    sem_g = (sem_g0, sem_g1)
    sem_s = (sem_s0, sem_s1)
    sem_w = (sem_w0, sem_w1)
    _zero_vmem_2d(zbuf, 16)
    rbase = pl.multiple_of(s * ROWS_PER_TILE, 8)
    for q in range(ROWS_PER_TILE // 16):
        pltpu.sync_copy(zbuf, acc.at[pl.ds(rbase + q * 16, 16)])
    plsc.subcore_barrier()

    nchunks = jnp.where(c == 0, NB0 // CH, NB1 // CH)
    start = jnp.where(c == 0, s * NB0, 16 * NB0 + s * NB1)

    def chunk(cc, _):
        cb = pl.multiple_of(start + cc * CH, 8)
        pltpu.sync_copy(g2_hbm.at[pl.ds(cb, CH)], gq)
        pltpu.sync_copy(k2_hbm.at[pl.ds(cb, CH)], kq)
        pltpu.sync_copy(d2_hbm.at[pl.ds(cb, CH)], dq)

        g_descs = [None, None]
        s_descs = [None, None]
        w_descs = [None, None]
        g_descs[0] = pltpu.async_copy(h_hbm.at[gq.at[0]], rows.at[0], sem_g[0])
        s_descs[0] = pltpu.async_copy(inv_hbm.at[kq.at[0]], sv.at[0], sem_s[0])
        for j in range(CH):
            p = j % 2
            q = 1 - p
            g_descs[p].wait()
            s_descs[p].wait()
            if j + 1 < CH:
                if j >= 1:
                    w_descs[q].wait()
                    w_descs[q] = None
                g_descs[q] = pltpu.async_copy(
                    h_hbm.at[gq.at[j + 1]], rows.at[q], sem_g[q])
                s_descs[q] = pltpu.async_copy(
                    inv_hbm.at[kq.at[j + 1]], sv.at[q], sem_s[q])
            _scale_block(rows.at[p], sv.at[p])
            w_descs[p] = pltpu.async_copy(
                rows.at[p], acc.at[dq.at[j]], sem_w[p], add=True)
        for d in w_descs:
            if d is not None:
                d.wait()
        return 0
    lax.fori_loop(0, nchunks, chunk, 0)
    plsc.subcore_barrier()
    for q in range(ROWS_PER_TILE // B):
        pltpu.sync_copy(acc.at[pl.ds(rbase + q * B, B)],
                        out_hbm.at[c, pl.ds(rbase + q * B, B)])


_agg_kernel = functools.partial(
    pl.kernel,
    out_type=jax.ShapeDtypeStruct((2, NP, D), jnp.float32),
    mesh=plsc.VectorSubcoreMesh(core_axis_name="c", subcore_axis_name="s",
                                num_cores=2, num_subcores=16),
    compiler_params=pltpu.CompilerParams(needs_layout_passes=False),
    scratch_types=[
        pltpu.VMEM((CH, B), jnp.int32),
        pltpu.VMEM((CH, B), jnp.int32),
        pltpu.VMEM((CH, B), jnp.int32),
        pltpu.VMEM((2, B, D), jnp.float32),
        pltpu.VMEM((2, B), jnp.float32),
        pltpu.VMEM((16, D), jnp.float32),
        pltpu.VMEM_SHARED((NP, D), jnp.float32),
        pltpu.SemaphoreType.DMA,
        pltpu.SemaphoreType.DMA,
        pltpu.SemaphoreType.DMA,
        pltpu.SemaphoreType.DMA,
        pltpu.SemaphoreType.DMA,
        pltpu.SemaphoreType.DMA,
    ],
)(_agg_body)


def _comb_body(p_ref, r_ref, o_ref):
    o_ref[...] = jnp.maximum(p_ref[0] + p_ref[1] + r_ref[...], 0.0)


def _combine(parts, root):
    return pl.pallas_call(
        _comb_body,
        grid=(NP // BN,),
        in_specs=[
            pl.BlockSpec((2, BN, D), lambda j: (0, j, 0)),
            pl.BlockSpec((BN, D), lambda j: (j, 0)),
        ],
        out_specs=pl.BlockSpec((BN, D), lambda j: (j, 0)),
        out_shape=jax.ShapeDtypeStruct((NP, D), jnp.float32),
    )(parts, root)


def _layer(xp, Ws, b, g3, k3, d3, inv):
    h = _transform(xp, Ws, b)
    h_flat = h.reshape((R + 1) * NP, D)
    parts = _agg_kernel(h_flat, g3, k3, d3, inv)
    return _combine(parts, h[R])


def kernel(x, edge_index, edge_type, W1, Wroot1, b1, W2, Wroot2, b2):
    src = edge_index[0].astype(jnp.int32)
    dst = edge_index[1].astype(jnp.int32)
    et = edge_type.astype(jnp.int32)

    g = et * NP + src
    k = dst * R + et
    pad = EPAD - E
    g3 = jnp.pad(g, (0, pad)).reshape(TOTBLK, B)
    k3 = jnp.pad(k, (0, pad), constant_values=SENT).reshape(TOTBLK, B)
    d3 = jnp.pad(dst, (0, pad)).reshape(TOTBLK, B)

    xp = jnp.pad(x, ((0, NP - N), (0, 0)))
    W1s = jnp.concatenate([W1, Wroot1[None]], axis=0)
    W2s = jnp.concatenate([W2, Wroot2[None]], axis=0)

    cnt = _count_kernel(k3).reshape(2, KT)
    inv = _inv_counts(cnt)

    h1 = _layer(xp, W1s, b1, g3, k3, d3, inv)
    h2 = _layer(h1, W2s, b2, g3, k3, d3, inv)
    return h2[:N]
